# baseline (speedup 1.0000x reference)
import functools

import jax
import jax.numpy as jnp
import numpy as np
from jax import lax
from jax.experimental import pallas as pl
from jax.experimental.pallas import tpu as pltpu

N = 100000
E = 1600000
B = 100
NPG = 1000
K = 50
D = 21
FEAT = 1280
NB_GOS = 5000
HID = FEAT + D


def _head_body(z_ref, wm1_ref, bm1_ref, g1_ref, be1_ref, wm2_ref, bm2_ref,
               g2_ref, be2_ref, wout_ref, bout_ref, out_ref):
    def ln(v, g, b):
        mu = jnp.mean(v, axis=-1, keepdims=True)
        var = jnp.mean((v - mu) ** 2, axis=-1, keepdims=True)
        return (v - mu) * lax.rsqrt(var + 1e-5) * g + b

    z = z_ref[...]
    z1 = ln(jax.nn.relu(
        jnp.dot(z, wm1_ref[...], preferred_element_type=jnp.float32)
        + bm1_ref[...]), g1_ref[...], be1_ref[...])
    z2 = z1 + ln(jax.nn.relu(
        jnp.dot(z1, wm2_ref[...], preferred_element_type=jnp.float32)
        + bm2_ref[...]), g2_ref[...], be2_ref[...])
    out_ref[...] = jax.nn.sigmoid(
        jnp.dot(z2, wout_ref[...], preferred_element_type=jnp.float32)
        + bout_ref[...])


def _head(z, Wm1, bm1, g1, be1, Wm2, bm2, g2, be2, Wout, bout):
    return pl.pallas_call(
        _head_body,
        out_shape=jax.ShapeDtypeStruct((B, NB_GOS), jnp.float32),
    )(z, Wm1, bm1, g1, be1, Wm2, bm2, g2, be2, Wout, bout)


def kernel(features, h, edge_index, W1, b1, Ws, bs, Wm1, bm1, g1, be1,
           Wm2, bm2, g2, be2, Wout, bout):
    src = edge_index[0]
    dst = edge_index[1]
    deg_out = jnp.maximum(jnp.zeros((N,), jnp.float32).at[src].add(1.0), 1.0)
    deg_in = jnp.maximum(jnp.zeros((N,), jnp.float32).at[dst].add(1.0), 1.0)
    x = h * (deg_out ** -0.5)[:, None]
    agg = jnp.zeros((N, D), jnp.float32).at[dst].add(x[src])
    x = agg * (deg_in ** -0.5)[:, None] @ W1 + b1
    y = (x @ Ws)[:, 0] * (deg_out ** -0.5)
    sagg = jnp.zeros((N,), jnp.float32).at[dst].add(y[src])
    score = sagg * (deg_in ** -0.5) + bs[0]

    dense = score.reshape(B, NPG)
    perm = jnp.argsort(-dense, axis=-1)[:, :K]
    perm = (perm + jnp.arange(B)[:, None] * NPG).reshape(-1)
    feat = x[perm] * jnp.tanh(score[perm])[:, None]
    pooled = feat.reshape(B, K, D).mean(axis=1)
    z = jnp.concatenate([features, pooled], axis=1)
    return _head(z, Wm1, bm1, g1, be1, Wm2, bm2, g2, be2, Wout, bout)
